# Initial kernel scaffold; baseline (speedup 1.0000x reference)
import functools

import jax
import jax.numpy as jnp
from jax import lax
from jax.experimental import pallas as pl
from jax.experimental.pallas import tpu as pltpu
from jax.experimental.pallas import tpu_sc as plsc

N = 100000
E = 3200000
IN_C = 5
HID_C = 16
OUT_C = 64
NUM_OUT = 10

NC = 2
NS = 16
NW = NC * NS
EW = E // NW
BA = 10000
BC = 5000

_SC_MESH = plsc.VectorSubcoreMesh(core_axis_name="c", subcore_axis_name="s")


@functools.partial(
    pl.kernel,
    out_type=jax.ShapeDtypeStruct((NC, N), jnp.float32),
    mesh=_SC_MESH,
    scratch_types=[
        pltpu.VMEM((BA,), jnp.int32),
        pltpu.VMEM((BA,), jnp.float32),
        pltpu.VMEM_SHARED((N,), jnp.float32),
    ],
)
def _sc_degree(dst_hbm, zeros_hbm, ones_hbm, deg_out, idx_v, ones_v, deg_sp):
    cid = lax.axis_index("c")
    sid = lax.axis_index("s")
    wid = cid * NS + sid

    @pl.when(sid == 0)
    def _():
        pltpu.sync_copy(zeros_hbm, deg_sp)

    pltpu.sync_copy(ones_hbm, ones_v)
    plsc.subcore_barrier()

    def body(i, carry):
        base = wid * EW + i * BA
        pltpu.sync_copy(dst_hbm.at[pl.ds(base, BA)], idx_v)
        pltpu.sync_copy(ones_v, deg_sp.at[idx_v], add=True)
        return carry

    lax.fori_loop(0, EW // BA, body, 0)
    plsc.subcore_barrier()

    @pl.when(sid < 4)
    def _():
        q = N // 4
        pltpu.sync_copy(deg_sp.at[pl.ds(sid * q, q)], deg_out.at[cid, pl.ds(sid * q, q)])


@functools.partial(
    pl.kernel,
    out_type=[
        jax.ShapeDtypeStruct((NC, N, HID_C), jnp.float32),
        jax.ShapeDtypeStruct((NC, N), jnp.float32),
    ],
    mesh=_SC_MESH,
    scratch_types=[
        pltpu.VMEM((BC,), jnp.int32),
        pltpu.VMEM((BC,), jnp.int32),
        pltpu.VMEM((BC, HID_C), jnp.float32),
        pltpu.VMEM((BC,), jnp.float32),
        pltpu.SemaphoreType.DMA,
        pltpu.VMEM_SHARED((N, HID_C), jnp.float32),
        pltpu.VMEM_SHARED((N,), jnp.float32),
    ],
)
def _sc_edges(src_hbm, dst_hbm, y_hbm, norm_hbm, agg_out, dn_out,
              sidx, didx, rows, nvals, sem, agg_sp, dn_sp):
    cid = lax.axis_index("c")
    sid = lax.axis_index("s")
    wid = cid * NS + sid

    @pl.when(sid == 0)
    def _():
        pltpu.sync_copy(y_hbm, agg_sp)
        pltpu.sync_copy(norm_hbm, dn_sp)

    plsc.subcore_barrier()

    def body(i, carry):
        base = wid * EW + i * BC
        pltpu.sync_copy(src_hbm.at[pl.ds(base, BC)], sidx)
        pltpu.sync_copy(dst_hbm.at[pl.ds(base, BC)], didx)
        pltpu.async_copy(y_hbm.at[sidx], rows, sem).wait()
        pltpu.sync_copy(rows, agg_sp.at[didx], add=True)
        pltpu.async_copy(norm_hbm.at[didx], nvals, sem).wait()
        pltpu.sync_copy(nvals, dn_sp.at[sidx], add=True)
        return carry

    lax.fori_loop(0, EW // BC, body, 0)
    plsc.subcore_barrier()

    rpt = N // NS
    pltpu.sync_copy(agg_sp.at[pl.ds(sid * rpt, rpt), :],
                    agg_out.at[cid, pl.ds(sid * rpt, rpt), :])

    @pl.when(sid < 4)
    def _():
        q = N // 4
        pltpu.sync_copy(dn_sp.at[pl.ds(sid * q, q)], dn_out.at[cid, pl.ds(sid * q, q)])


NB = 10000


def _tc_prep_body(degp, xb, w1, yout, nout):
    deg = degp[0] + degp[1] + 1.0
    nrm = lax.rsqrt(deg)
    z = jnp.dot(xb[...], w1[...], preferred_element_type=jnp.float32)
    yout[...] = z * nrm[:, None]
    nout[...] = nrm


def _tc_prep(deg_part, x, W1):
    return pl.pallas_call(
        _tc_prep_body,
        grid=(N // NB,),
        in_specs=[
            pl.BlockSpec((NC, NB), lambda i: (0, i)),
            pl.BlockSpec((NB, IN_C), lambda i: (i, 0)),
            pl.BlockSpec((IN_C, HID_C), lambda i: (0, 0)),
        ],
        out_specs=[
            pl.BlockSpec((NB, HID_C), lambda i: (i, 0)),
            pl.BlockSpec((NB,), lambda i: (i,)),
        ],
        out_shape=[
            jax.ShapeDtypeStruct((N, HID_C), jnp.float32),
            jax.ShapeDtypeStruct((N,), jnp.float32),
        ],
    )(deg_part, x, W1)


def _tc_final_body(ap, yb, dnp, nb_, b1r, w2r, b2r, war, bar, out, acc):
    i = pl.program_id(0)

    @pl.when(i == 0)
    def _():
        acc[...] = jnp.zeros_like(acc)

    agg = ap[0] + ap[1] - yb[...]
    dn = dnp[0] + dnp[1] - nb_[...]
    nrm = nb_[...]
    h = jnp.maximum(b1r[...][None, :] + nrm[:, None] * agg, 0.0)
    w = nrm * dn
    acc[...] += jnp.sum(h * w[:, None], axis=0)[None, :]

    @pl.when(i == pl.num_programs(0) - 1)
    def _():
        s16 = acc[...]
        sumh2 = jnp.dot(s16, w2r[...], preferred_element_type=jnp.float32)
        feat = (sumh2 + float(N) * b2r[...][None, :]) * (1.0 / float(N))
        out[...] = jnp.dot(feat, war[...], preferred_element_type=jnp.float32) + bar[...][None, :]


def _tc_final(agg_part, y, dn_part, norm, b1, W2, b2, Wa, ba):
    return pl.pallas_call(
        _tc_final_body,
        grid=(N // NB,),
        in_specs=[
            pl.BlockSpec((NC, NB, HID_C), lambda i: (0, i, 0)),
            pl.BlockSpec((NB, HID_C), lambda i: (i, 0)),
            pl.BlockSpec((NC, NB), lambda i: (0, i)),
            pl.BlockSpec((NB,), lambda i: (i,)),
            pl.BlockSpec((HID_C,), lambda i: (0,)),
            pl.BlockSpec((HID_C, OUT_C), lambda i: (0, 0)),
            pl.BlockSpec((OUT_C,), lambda i: (0,)),
            pl.BlockSpec((OUT_C, NUM_OUT), lambda i: (0, 0)),
            pl.BlockSpec((NUM_OUT,), lambda i: (0,)),
        ],
        out_specs=pl.BlockSpec((1, NUM_OUT), lambda i: (0, 0)),
        out_shape=jax.ShapeDtypeStruct((1, NUM_OUT), jnp.float32),
        scratch_shapes=[pltpu.VMEM((1, HID_C), jnp.float32)],
    )(agg_part, y, dn_part, norm, b1, W2, b2, Wa, ba)


@jax.jit
def kernel(x, edge_index, W1, b1, W2, b2, Wa, ba):
    src = edge_index[0]
    dst = edge_index[1]
    zeros_n = jnp.zeros((N,), jnp.float32)
    ones_b = jnp.ones((BA,), jnp.float32)

    deg_part = _sc_degree(dst, zeros_n, ones_b)
    y, norm = _tc_prep(deg_part, x, W1)
    agg_part, dn_part = _sc_edges(src, dst, y, norm)
    return _tc_final(agg_part, y, dn_part, norm, b1, W2, b2, Wa, ba)


# speedup vs baseline: 68.9775x; 68.9775x over previous
import functools

import jax
import jax.numpy as jnp
from jax import lax
from jax.experimental import pallas as pl
from jax.experimental.pallas import tpu as pltpu
from jax.experimental.pallas import tpu_sc as plsc

N = 100000
E = 3200000
IN_C = 5
HID_C = 16
OUT_C = 64
NUM_OUT = 10

NC = 2
NS = 16
NW = NC * NS
EW = E // NW
BA = 10000
BC = 1000

_SC_MESH = plsc.VectorSubcoreMesh(core_axis_name="c", subcore_axis_name="s")
_SC_PARAMS = pltpu.CompilerParams(use_tc_tiling_on_sc=False)


@functools.partial(
    pl.kernel,
    out_type=jax.ShapeDtypeStruct((NC * N,), jnp.float32),
    mesh=_SC_MESH,
    scratch_types=[
        pltpu.VMEM((BA,), jnp.int32),
        pltpu.VMEM((BA,), jnp.float32),
        pltpu.VMEM_SHARED((N,), jnp.float32),
    ],
    compiler_params=_SC_PARAMS,
)
def _sc_degree(dst_hbm, zeros_hbm, ones_hbm, deg_out, idx_v, ones_v, deg_sp):
    cid = lax.axis_index("c")
    sid = lax.axis_index("s")
    wid = cid * NS + sid

    @pl.when(sid == 0)
    def _():
        pltpu.sync_copy(zeros_hbm, deg_sp)

    pltpu.sync_copy(ones_hbm, ones_v)
    plsc.subcore_barrier()

    def body(i, carry):
        base = wid * EW + i * BA
        pltpu.sync_copy(dst_hbm.at[pl.ds(base, BA)], idx_v)
        pltpu.sync_copy(ones_v, deg_sp.at[idx_v], add=True)
        return carry

    lax.fori_loop(0, EW // BA, body, 0)
    plsc.subcore_barrier()

    @pl.when(sid < N // BA)
    def _():
        pltpu.sync_copy(deg_sp.at[pl.ds(sid * BA, BA)], ones_v)
        pltpu.sync_copy(ones_v, deg_out.at[pl.ds(cid * N + sid * BA, BA)])


@functools.partial(
    pl.kernel,
    out_type=[
        jax.ShapeDtypeStruct((NC * N, HID_C), jnp.float32),
        jax.ShapeDtypeStruct((NC * N,), jnp.float32),
    ],
    mesh=_SC_MESH,
    scratch_types=[
        pltpu.VMEM((BC,), jnp.int32),
        pltpu.VMEM((BC,), jnp.int32),
        pltpu.VMEM((BC, HID_C), jnp.float32),
        pltpu.VMEM((BC,), jnp.float32),
        pltpu.SemaphoreType.DMA,
        pltpu.VMEM_SHARED((N, HID_C), jnp.float32),
        pltpu.VMEM_SHARED((N,), jnp.float32),
    ],
    compiler_params=_SC_PARAMS,
)
def _sc_edges(src_hbm, dst_hbm, y_hbm, norm_hbm, agg_out, dn_out,
              sidx, didx, rows, nvals, sem, agg_sp, dn_sp):
    cid = lax.axis_index("c")
    sid = lax.axis_index("s")
    wid = cid * NS + sid

    @pl.when(sid == 0)
    def _():
        pltpu.sync_copy(y_hbm, agg_sp)
        pltpu.sync_copy(norm_hbm, dn_sp)

    plsc.subcore_barrier()

    def body(i, carry):
        base = wid * EW + i * BC
        pltpu.sync_copy(src_hbm.at[pl.ds(base, BC)], sidx)
        pltpu.sync_copy(dst_hbm.at[pl.ds(base, BC)], didx)
        pltpu.async_copy(y_hbm.at[sidx], rows, sem).wait()
        pltpu.sync_copy(rows, agg_sp.at[didx], add=True)
        pltpu.async_copy(norm_hbm.at[didx], nvals, sem).wait()
        pltpu.sync_copy(nvals, dn_sp.at[sidx], add=True)
        return carry

    lax.fori_loop(0, EW // BC, body, 0)
    plsc.subcore_barrier()

    nch = N // BC

    def wb(j, carry):
        k = sid + j * NS

        @pl.when(k < nch)
        def _():
            pltpu.sync_copy(agg_sp.at[pl.ds(k * BC, BC), :], rows)
            pltpu.sync_copy(rows, agg_out.at[pl.ds(cid * N + k * BC, BC), :])
            pltpu.sync_copy(dn_sp.at[pl.ds(k * BC, BC)], nvals)
            pltpu.sync_copy(nvals, dn_out.at[pl.ds(cid * N + k * BC, BC)])

        return carry

    lax.fori_loop(0, (nch + NS - 1) // NS, wb, 0)


NB = 2000


def _tc_prep_body(degp, xb, w1, yout, nout):
    deg = degp[0] + degp[1] + 1.0
    nrm = lax.rsqrt(deg)
    z = jnp.dot(xb[...], w1[...], preferred_element_type=jnp.float32)
    yout[...] = z * nrm
    nout[...] = nrm


def _tc_prep(deg_part, x, W1):
    return pl.pallas_call(
        _tc_prep_body,
        grid=(N // NB,),
        in_specs=[
            pl.BlockSpec((NC, NB, 1), lambda i: (0, i, 0)),
            pl.BlockSpec((NB, IN_C), lambda i: (i, 0)),
            pl.BlockSpec((IN_C, HID_C), lambda i: (0, 0)),
        ],
        out_specs=[
            pl.BlockSpec((NB, HID_C), lambda i: (i, 0)),
            pl.BlockSpec((NB, 1), lambda i: (i, 0)),
        ],
        out_shape=[
            jax.ShapeDtypeStruct((N, HID_C), jnp.float32),
            jax.ShapeDtypeStruct((N, 1), jnp.float32),
        ],
    )(deg_part, x, W1)


def _tc_final_body(ap, yb, dnp, nb_, b1r, w2r, b2r, war, bar, out, acc):
    i = pl.program_id(0)

    @pl.when(i == 0)
    def _():
        acc[...] = jnp.zeros_like(acc)

    agg = ap[0] + ap[1] - yb[...]
    nrm = nb_[...]
    dn = dnp[0] + dnp[1] - nrm
    h = jnp.maximum(b1r[...][None, :] + nrm * agg, 0.0)
    w = nrm * dn
    acc[...] += jnp.sum(h * w, axis=0)[None, :]

    @pl.when(i == pl.num_programs(0) - 1)
    def _():
        s16 = acc[...]
        sumh2 = jnp.dot(s16, w2r[...], preferred_element_type=jnp.float32)
        feat = (sumh2 + float(N) * b2r[...][None, :]) * (1.0 / float(N))
        out[...] = jnp.dot(feat, war[...], preferred_element_type=jnp.float32) + bar[...][None, :]


def _tc_final(agg_part, y, dn_part, norm, b1, W2, b2, Wa, ba):
    return pl.pallas_call(
        _tc_final_body,
        grid=(N // NB,),
        in_specs=[
            pl.BlockSpec((NC, NB, HID_C), lambda i: (0, i, 0)),
            pl.BlockSpec((NB, HID_C), lambda i: (i, 0)),
            pl.BlockSpec((NC, NB, 1), lambda i: (0, i, 0)),
            pl.BlockSpec((NB, 1), lambda i: (i, 0)),
            pl.BlockSpec((HID_C,), lambda i: (0,)),
            pl.BlockSpec((HID_C, OUT_C), lambda i: (0, 0)),
            pl.BlockSpec((OUT_C,), lambda i: (0,)),
            pl.BlockSpec((OUT_C, NUM_OUT), lambda i: (0, 0)),
            pl.BlockSpec((NUM_OUT,), lambda i: (0,)),
        ],
        out_specs=pl.BlockSpec((1, NUM_OUT), lambda i: (0, 0)),
        out_shape=jax.ShapeDtypeStruct((1, NUM_OUT), jnp.float32),
        scratch_shapes=[pltpu.VMEM((1, HID_C), jnp.float32)],
    )(agg_part, y, dn_part, norm, b1, W2, b2, Wa, ba)


@jax.jit
def kernel(x, edge_index, W1, b1, W2, b2, Wa, ba):
    src = edge_index[0]
    dst = edge_index[1]
    zeros_n = jnp.zeros((N,), jnp.float32)
    ones_b = jnp.ones((BA,), jnp.float32)

    deg_part = _sc_degree(dst, zeros_n, ones_b)
    y, norm2d = _tc_prep(deg_part.reshape(NC, N, 1), x, W1)
    agg_part, dn_part = _sc_edges(src, dst, y, norm2d.reshape(N))
    return _tc_final(agg_part.reshape(NC, N, HID_C), y,
                     dn_part.reshape(NC, N, 1), norm2d, b1, W2, b2, Wa, ba)


# speedup vs baseline: 87.8028x; 1.2729x over previous
import functools

import jax
import jax.numpy as jnp
from jax import lax
from jax.experimental import pallas as pl
from jax.experimental.pallas import tpu as pltpu
from jax.experimental.pallas import tpu_sc as plsc

N = 100000
E = 3200000
IN_C = 5
HID_C = 16
OUT_C = 64
NUM_OUT = 10

NC = 2
NS = 16
NW = NC * NS
EW = E // NW
BA = 10000
BC = 1000

_SC_MESH = plsc.VectorSubcoreMesh(core_axis_name="c", subcore_axis_name="s")
_SC_PARAMS = pltpu.CompilerParams(use_tc_tiling_on_sc=False)


@functools.partial(
    pl.kernel,
    out_type=jax.ShapeDtypeStruct((NC * N,), jnp.float32),
    mesh=_SC_MESH,
    scratch_types=[
        pltpu.VMEM((BA,), jnp.int32),
        pltpu.VMEM((BA,), jnp.float32),
        pltpu.VMEM_SHARED((N,), jnp.float32),
    ],
    compiler_params=_SC_PARAMS,
)
def _sc_degree(dst_hbm, zeros_hbm, ones_hbm, deg_out, idx_v, ones_v, deg_sp):
    cid = lax.axis_index("c")
    sid = lax.axis_index("s")
    wid = cid * NS + sid

    @pl.when(sid == 0)
    def _():
        pltpu.sync_copy(zeros_hbm, deg_sp)

    pltpu.sync_copy(ones_hbm, ones_v)
    plsc.subcore_barrier()

    def body(i, carry):
        base = wid * EW + i * BA
        pltpu.sync_copy(dst_hbm.at[pl.ds(base, BA)], idx_v)
        pltpu.sync_copy(ones_v, deg_sp.at[idx_v], add=True)
        return carry

    lax.fori_loop(0, EW // BA, body, 0)
    plsc.subcore_barrier()

    @pl.when(sid < N // BA)
    def _():
        pltpu.sync_copy(deg_sp.at[pl.ds(sid * BA, BA)], ones_v)
        pltpu.sync_copy(ones_v, deg_out.at[pl.ds(cid * N + sid * BA, BA)])


SB = 4000
BCH = 400
NSB = EW // SB
NCH = SB // BCH


@functools.partial(
    pl.kernel,
    out_type=jax.ShapeDtypeStruct((NC * N, HID_C), jnp.float32),
    mesh=_SC_MESH,
    scratch_types=[
        pltpu.VMEM((SB,), jnp.int32),
        pltpu.VMEM((SB,), jnp.int32),
        pltpu.VMEM((BCH, HID_C), jnp.float32),
        pltpu.VMEM((BCH, HID_C), jnp.float32),
        pltpu.SemaphoreType.DMA,
        pltpu.SemaphoreType.DMA,
        pltpu.SemaphoreType.DMA,
        pltpu.SemaphoreType.DMA,
        pltpu.VMEM_SHARED((N, HID_C), jnp.float32),
    ],
    compiler_params=_SC_PARAMS,
)
def _sc_agg(src_hbm, dst_hbm, y_hbm, agg_out,
            sidx, didx, rows0, rows1, sg0, sg1, ss0, ss1, agg_sp):
    cid = lax.axis_index("c")
    sid = lax.axis_index("s")
    wid = cid * NS + sid
    rows = (rows0, rows1)
    sg = (sg0, sg1)
    ss = (ss0, ss1)

    @pl.when(sid == 0)
    def _():
        pltpu.sync_copy(y_hbm, agg_sp)

    plsc.subcore_barrier()

    def sb_body(s, carry):
        base = wid * EW + s * SB
        pltpu.sync_copy(src_hbm.at[pl.ds(base, SB)], sidx)
        pltpu.sync_copy(dst_hbm.at[pl.ds(base, SB)], didx)
        g = [None, None]
        sc = [None, None]
        g[0] = pltpu.async_copy(y_hbm.at[sidx.at[pl.ds(0, BCH)]], rows[0], sg[0])
        for j in range(NCH):
            b = j & 1
            nb = b ^ 1
            g[b].wait()
            if j + 1 < NCH:
                if sc[nb] is not None:
                    sc[nb].wait()
                g[nb] = pltpu.async_copy(
                    y_hbm.at[sidx.at[pl.ds((j + 1) * BCH, BCH)]], rows[nb], sg[nb])
            sc[b] = pltpu.async_copy(
                rows[b], agg_sp.at[didx.at[pl.ds(j * BCH, BCH)]], ss[b], add=True)
        sc[0].wait()
        sc[1].wait()
        return carry

    lax.fori_loop(0, NSB, sb_body, 0)
    plsc.subcore_barrier()

    nch = N // BCH

    def wb(j, carry):
        k = sid + j * NS

        @pl.when(k < nch)
        def _():
            pltpu.sync_copy(agg_sp.at[pl.ds(k * BCH, BCH), :], rows0)
            pltpu.sync_copy(rows0, agg_out.at[pl.ds(cid * N + k * BCH, BCH), :])

        return carry

    lax.fori_loop(0, (nch + NS - 1) // NS, wb, 0)


SB2 = 25000
BC2 = 1000
NSB2 = EW // SB2
NCH2 = SB2 // BC2


@functools.partial(
    pl.kernel,
    out_type=jax.ShapeDtypeStruct((NC * N,), jnp.float32),
    mesh=_SC_MESH,
    scratch_types=[
        pltpu.VMEM((SB2,), jnp.int32),
        pltpu.VMEM((SB2,), jnp.int32),
        pltpu.VMEM((BC2,), jnp.float32),
        pltpu.VMEM((BC2,), jnp.float32),
        pltpu.SemaphoreType.DMA,
        pltpu.SemaphoreType.DMA,
        pltpu.SemaphoreType.DMA,
        pltpu.SemaphoreType.DMA,
        pltpu.VMEM_SHARED((N,), jnp.float32),
        pltpu.VMEM_SHARED((N,), jnp.float32),
    ],
    compiler_params=_SC_PARAMS,
)
def _sc_dnorm(src_hbm, dst_hbm, norm_hbm, dn_out,
              sidx, didx, nv0, nv1, sg0, sg1, ss0, ss1, dn_sp, norm_sp):
    cid = lax.axis_index("c")
    sid = lax.axis_index("s")
    wid = cid * NS + sid
    nv = (nv0, nv1)
    sg = (sg0, sg1)
    ss = (ss0, ss1)

    @pl.when(sid == 0)
    def _():
        pltpu.sync_copy(norm_hbm, dn_sp)
        pltpu.sync_copy(norm_hbm, norm_sp)

    plsc.subcore_barrier()

    def sb_body(s, carry):
        base = wid * EW + s * SB2
        pltpu.sync_copy(src_hbm.at[pl.ds(base, SB2)], sidx)
        pltpu.sync_copy(dst_hbm.at[pl.ds(base, SB2)], didx)
        g = [None, None]
        sc = [None, None]
        g[0] = pltpu.async_copy(norm_sp.at[didx.at[pl.ds(0, BC2)]], nv[0], sg[0])
        for j in range(NCH2):
            b = j & 1
            nb = b ^ 1
            g[b].wait()
            if j + 1 < NCH2:
                if sc[nb] is not None:
                    sc[nb].wait()
                g[nb] = pltpu.async_copy(
                    norm_sp.at[didx.at[pl.ds((j + 1) * BC2, BC2)]], nv[nb], sg[nb])
            sc[b] = pltpu.async_copy(
                nv[b], dn_sp.at[sidx.at[pl.ds(j * BC2, BC2)]], ss[b], add=True)
        sc[0].wait()
        sc[1].wait()
        return carry

    lax.fori_loop(0, NSB2, sb_body, 0)
    plsc.subcore_barrier()

    nch = N // BC2

    def wb(j, carry):
        k = sid + j * NS

        @pl.when(k < nch)
        def _():
            pltpu.sync_copy(dn_sp.at[pl.ds(k * BC2, BC2)], nv0)
            pltpu.sync_copy(nv0, dn_out.at[pl.ds(cid * N + k * BC2, BC2)])

        return carry

    lax.fori_loop(0, (nch + NS - 1) // NS, wb, 0)


NB = 2000


def _tc_prep_body(degp, xb, w1, yout, nout):
    deg = degp[0] + degp[1] + 1.0
    nrm = lax.rsqrt(deg)
    z = jnp.dot(xb[...], w1[...], preferred_element_type=jnp.float32)
    yout[...] = z * nrm
    nout[...] = nrm


def _tc_prep(deg_part, x, W1):
    return pl.pallas_call(
        _tc_prep_body,
        grid=(N // NB,),
        in_specs=[
            pl.BlockSpec((NC, NB, 1), lambda i: (0, i, 0)),
            pl.BlockSpec((NB, IN_C), lambda i: (i, 0)),
            pl.BlockSpec((IN_C, HID_C), lambda i: (0, 0)),
        ],
        out_specs=[
            pl.BlockSpec((NB, HID_C), lambda i: (i, 0)),
            pl.BlockSpec((NB, 1), lambda i: (i, 0)),
        ],
        out_shape=[
            jax.ShapeDtypeStruct((N, HID_C), jnp.float32),
            jax.ShapeDtypeStruct((N, 1), jnp.float32),
        ],
    )(deg_part, x, W1)


def _tc_final_body(ap, yb, dnp, nb_, b1r, w2r, b2r, war, bar, out, acc):
    i = pl.program_id(0)

    @pl.when(i == 0)
    def _():
        acc[...] = jnp.zeros_like(acc)

    agg = ap[0] + ap[1] - yb[...]
    nrm = nb_[...]
    dn = dnp[0] + dnp[1] - nrm
    h = jnp.maximum(b1r[...][None, :] + nrm * agg, 0.0)
    w = nrm * dn
    acc[...] += jnp.sum(h * w, axis=0)[None, :]

    @pl.when(i == pl.num_programs(0) - 1)
    def _():
        s16 = acc[...]
        sumh2 = jnp.dot(s16, w2r[...], preferred_element_type=jnp.float32)
        feat = (sumh2 + float(N) * b2r[...][None, :]) * (1.0 / float(N))
        out[...] = jnp.dot(feat, war[...], preferred_element_type=jnp.float32) + bar[...][None, :]


def _tc_final(agg_part, y, dn_part, norm, b1, W2, b2, Wa, ba):
    return pl.pallas_call(
        _tc_final_body,
        grid=(N // NB,),
        in_specs=[
            pl.BlockSpec((NC, NB, HID_C), lambda i: (0, i, 0)),
            pl.BlockSpec((NB, HID_C), lambda i: (i, 0)),
            pl.BlockSpec((NC, NB, 1), lambda i: (0, i, 0)),
            pl.BlockSpec((NB, 1), lambda i: (i, 0)),
            pl.BlockSpec((HID_C,), lambda i: (0,)),
            pl.BlockSpec((HID_C, OUT_C), lambda i: (0, 0)),
            pl.BlockSpec((OUT_C,), lambda i: (0,)),
            pl.BlockSpec((OUT_C, NUM_OUT), lambda i: (0, 0)),
            pl.BlockSpec((NUM_OUT,), lambda i: (0,)),
        ],
        out_specs=pl.BlockSpec((1, NUM_OUT), lambda i: (0, 0)),
        out_shape=jax.ShapeDtypeStruct((1, NUM_OUT), jnp.float32),
        scratch_shapes=[pltpu.VMEM((1, HID_C), jnp.float32)],
    )(agg_part, y, dn_part, norm, b1, W2, b2, Wa, ba)


@jax.jit
def kernel(x, edge_index, W1, b1, W2, b2, Wa, ba):
    src = edge_index[0]
    dst = edge_index[1]
    zeros_n = jnp.zeros((N,), jnp.float32)
    ones_b = jnp.ones((BA,), jnp.float32)

    deg_part = _sc_degree(dst, zeros_n, ones_b)
    y, norm2d = _tc_prep(deg_part.reshape(NC, N, 1), x, W1)
    agg_part = _sc_agg(src, dst, y)
    dn_part = _sc_dnorm(src, dst, norm2d.reshape(N))
    return _tc_final(agg_part.reshape(NC, N, HID_C), y,
                     dn_part.reshape(NC, N, 1), norm2d, b1, W2, b2, Wa, ba)


# speedup vs baseline: 123.6605x; 1.4084x over previous
import functools

import jax
import jax.numpy as jnp
from jax import lax
from jax.experimental import pallas as pl
from jax.experimental.pallas import tpu as pltpu
from jax.experimental.pallas import tpu_sc as plsc

N = 100000
E = 3200000
IN_C = 5
HID_C = 16
OUT_C = 64
NUM_OUT = 10

NC = 2
NS = 16
NW = NC * NS
EW = E // NW
BA = 10000
BC = 1000

_SC_MESH = plsc.VectorSubcoreMesh(core_axis_name="c", subcore_axis_name="s")
_SC_PARAMS = pltpu.CompilerParams(use_tc_tiling_on_sc=False)


@functools.partial(
    pl.kernel,
    out_type=jax.ShapeDtypeStruct((NC * N,), jnp.float32),
    mesh=_SC_MESH,
    scratch_types=[
        pltpu.VMEM((BA,), jnp.int32),
        pltpu.VMEM((BA,), jnp.float32),
        pltpu.VMEM_SHARED((N,), jnp.float32),
    ],
    compiler_params=_SC_PARAMS,
)
def _sc_degree(dst_hbm, zeros_hbm, ones_hbm, deg_out, idx_v, ones_v, deg_sp):
    cid = lax.axis_index("c")
    sid = lax.axis_index("s")
    wid = cid * NS + sid

    @pl.when(sid == 0)
    def _():
        pltpu.sync_copy(zeros_hbm, deg_sp)

    pltpu.sync_copy(ones_hbm, ones_v)
    plsc.subcore_barrier()

    def body(i, carry):
        base = wid * EW + i * BA
        pltpu.sync_copy(dst_hbm.at[pl.ds(base, BA)], idx_v)
        pltpu.sync_copy(ones_v, deg_sp.at[idx_v], add=True)
        return carry

    lax.fori_loop(0, EW // BA, body, 0)
    plsc.subcore_barrier()

    @pl.when(sid < N // BA)
    def _():
        pltpu.sync_copy(deg_sp.at[pl.ds(sid * BA, BA)], ones_v)
        pltpu.sync_copy(ones_v, deg_out.at[pl.ds(cid * N + sid * BA, BA)])


SB = 4000
BCH = 400
NSB = EW // SB
NCH = SB // BCH


@functools.partial(
    pl.kernel,
    out_type=jax.ShapeDtypeStruct((NC * N, HID_C), jnp.float32),
    mesh=_SC_MESH,
    scratch_types=[
        pltpu.VMEM((SB,), jnp.int32),
        pltpu.VMEM((SB,), jnp.int32),
        pltpu.VMEM((BCH, HID_C), jnp.float32),
        pltpu.VMEM((BCH, HID_C), jnp.float32),
        pltpu.SemaphoreType.DMA,
        pltpu.SemaphoreType.DMA,
        pltpu.SemaphoreType.DMA,
        pltpu.SemaphoreType.DMA,
        pltpu.VMEM_SHARED((N, HID_C), jnp.float32),
    ],
    compiler_params=_SC_PARAMS,
)
def _sc_agg(src_hbm, dst_hbm, y_hbm, agg_out,
            sidx, didx, rows0, rows1, sg0, sg1, ss0, ss1, agg_sp):
    cid = lax.axis_index("c")
    sid = lax.axis_index("s")
    wid = cid * NS + sid
    rows = (rows0, rows1)
    sg = (sg0, sg1)
    ss = (ss0, ss1)

    @pl.when(sid == 0)
    def _():
        pltpu.sync_copy(y_hbm, agg_sp)

    plsc.subcore_barrier()

    def sb_body(s, carry):
        base = wid * EW + s * SB
        pltpu.sync_copy(src_hbm.at[pl.ds(base, SB)], sidx)
        pltpu.sync_copy(dst_hbm.at[pl.ds(base, SB)], didx)
        g = [None, None]
        sc = [None, None]
        g[0] = pltpu.async_copy(y_hbm.at[sidx.at[pl.ds(0, BCH)]], rows[0], sg[0])
        for j in range(NCH):
            b = j & 1
            nb = b ^ 1
            g[b].wait()
            if j + 1 < NCH:
                if sc[nb] is not None:
                    sc[nb].wait()
                g[nb] = pltpu.async_copy(
                    y_hbm.at[sidx.at[pl.ds((j + 1) * BCH, BCH)]], rows[nb], sg[nb])
            sc[b] = pltpu.async_copy(
                rows[b], agg_sp.at[didx.at[pl.ds(j * BCH, BCH)]], ss[b], add=True)
        sc[0].wait()
        sc[1].wait()
        return carry

    lax.fori_loop(0, NSB, sb_body, 0)
    plsc.subcore_barrier()

    nch = N // BCH

    def wb(j, carry):
        k = sid + j * NS

        @pl.when(k < nch)
        def _():
            pltpu.sync_copy(agg_sp.at[pl.ds(k * BCH, BCH), :], rows0)
            pltpu.sync_copy(rows0, agg_out.at[pl.ds(cid * N + k * BCH, BCH), :])

        return carry

    lax.fori_loop(0, (nch + NS - 1) // NS, wb, 0)


SB2 = 25000
BC2 = 1000
NSB2 = EW // SB2
NCH2 = SB2 // BC2


@functools.partial(
    pl.kernel,
    out_type=jax.ShapeDtypeStruct((NC * N,), jnp.float32),
    mesh=_SC_MESH,
    scratch_types=[
        pltpu.VMEM((SB2,), jnp.int32),
        pltpu.VMEM((SB2,), jnp.int32),
        pltpu.VMEM((BC2,), jnp.float32),
        pltpu.VMEM((BC2,), jnp.float32),
        pltpu.SemaphoreType.DMA,
        pltpu.SemaphoreType.DMA,
        pltpu.SemaphoreType.DMA,
        pltpu.SemaphoreType.DMA,
        pltpu.VMEM_SHARED((N,), jnp.float32),
        pltpu.VMEM_SHARED((N,), jnp.float32),
    ],
    compiler_params=_SC_PARAMS,
)
def _sc_dnorm(src_hbm, dst_hbm, norm_hbm, dn_out,
              sidx, didx, nv0, nv1, sg0, sg1, ss0, ss1, dn_sp, norm_sp):
    cid = lax.axis_index("c")
    sid = lax.axis_index("s")
    wid = cid * NS + sid
    nv = (nv0, nv1)
    sg = (sg0, sg1)
    ss = (ss0, ss1)

    @pl.when(sid == 0)
    def _():
        pltpu.sync_copy(norm_hbm, dn_sp)
        pltpu.sync_copy(norm_hbm, norm_sp)

    plsc.subcore_barrier()

    def sb_body(s, carry):
        base = wid * EW + s * SB2
        pltpu.sync_copy(src_hbm.at[pl.ds(base, SB2)], sidx)
        pltpu.sync_copy(dst_hbm.at[pl.ds(base, SB2)], didx)
        g = [None, None]
        sc = [None, None]
        g[0] = pltpu.async_copy(norm_sp.at[didx.at[pl.ds(0, BC2)]], nv[0], sg[0])
        for j in range(NCH2):
            b = j & 1
            nb = b ^ 1
            g[b].wait()
            if j + 1 < NCH2:
                if sc[nb] is not None:
                    sc[nb].wait()
                g[nb] = pltpu.async_copy(
                    norm_sp.at[didx.at[pl.ds((j + 1) * BC2, BC2)]], nv[nb], sg[nb])
            sc[b] = pltpu.async_copy(
                nv[b], dn_sp.at[sidx.at[pl.ds(j * BC2, BC2)]], ss[b], add=True)
        sc[0].wait()
        sc[1].wait()
        return carry

    lax.fori_loop(0, NSB2, sb_body, 0)
    plsc.subcore_barrier()

    nch = N // BC2

    def wb(j, carry):
        k = sid + j * NS

        @pl.when(k < nch)
        def _():
            pltpu.sync_copy(dn_sp.at[pl.ds(k * BC2, BC2)], nv0)
            pltpu.sync_copy(nv0, dn_out.at[pl.ds(cid * N + k * BC2, BC2)])

        return carry

    lax.fori_loop(0, (nch + NS - 1) // NS, wb, 0)


NBZ = 1000
GZ = N // NBZ


def _tc_z_body(xb, w1, zout):
    zout[...] = jnp.dot(xb[...], w1[...], preferred_element_type=jnp.float32)


def _tc_z(x, W1):
    return pl.pallas_call(
        _tc_z_body,
        grid=(GZ,),
        in_specs=[
            pl.BlockSpec((NBZ, IN_C), lambda i: (i, 0)),
            pl.BlockSpec((IN_C, HID_C), lambda i: (0, 0)),
        ],
        out_specs=pl.BlockSpec((NBZ, HID_C), lambda i: (i, 0)),
        out_shape=jax.ShapeDtypeStruct((N, HID_C), jnp.float32),
    )(x, W1)


CHK = 800
NCHK = N // CHK


def _newton_rsqrt(d):
    i = lax.bitcast_convert_type(d, jnp.int32)
    magic = jnp.full((16,), 0x5F3759DF, jnp.int32)
    r = lax.bitcast_convert_type(magic - (i >> 1), jnp.float32)
    for _ in range(3):
        r = r * (1.5 - 0.5 * d * r * r)
    return r


@functools.partial(
    pl.kernel,
    out_type=[
        jax.ShapeDtypeStruct((N, HID_C), jnp.float32),
        jax.ShapeDtypeStruct((N,), jnp.float32),
    ],
    mesh=_SC_MESH,
    scratch_types=[
        pltpu.VMEM((CHK,), jnp.float32),
        pltpu.VMEM((CHK,), jnp.float32),
        pltpu.VMEM((CHK, HID_C), jnp.float32),
        pltpu.VMEM((CHK,), jnp.float32),
    ],
    compiler_params=_SC_PARAMS,
)
def _sc_prep(degp_hbm, z_hbm, y_out, norm_out, d0v, d1v, zc, nc):
    cid = lax.axis_index("c")
    sid = lax.axis_index("s")
    wid = cid * NS + sid

    def chunk(j, carry):
        k = wid + j * NW

        @pl.when(k < NCHK)
        def _():
            base = k * CHK
            pltpu.sync_copy(degp_hbm.at[pl.ds(base, CHK)], d0v)
            pltpu.sync_copy(degp_hbm.at[pl.ds(N + base, CHK)], d1v)
            pltpu.sync_copy(z_hbm.at[pl.ds(base, CHK), :], zc)

            def grp(g, c2):
                sl = pl.ds(g * 16, 16)
                d = d0v[sl] + d1v[sl] + 1.0
                nr = _newton_rsqrt(d)
                nc[sl] = nr
                for r in range(16):
                    row = zc[g * 16 + r, :]
                    zc[g * 16 + r, :] = row * nr[r]
                return c2

            lax.fori_loop(0, CHK // 16, grp, 0)
            pltpu.sync_copy(nc, norm_out.at[pl.ds(base, CHK)])
            pltpu.sync_copy(zc, y_out.at[pl.ds(base, CHK), :])

        return carry

    lax.fori_loop(0, (NCHK + NW - 1) // NW, chunk, 0)


@functools.partial(
    pl.kernel,
    out_type=jax.ShapeDtypeStruct((NW * HID_C,), jnp.float32),
    mesh=_SC_MESH,
    scratch_types=[
        pltpu.VMEM((CHK, HID_C), jnp.float32),
        pltpu.VMEM((CHK, HID_C), jnp.float32),
        pltpu.VMEM((CHK, HID_C), jnp.float32),
        pltpu.VMEM((CHK,), jnp.float32),
        pltpu.VMEM((CHK,), jnp.float32),
        pltpu.VMEM((CHK,), jnp.float32),
        pltpu.VMEM((16,), jnp.float32),
        pltpu.VMEM((16,), jnp.float32),
    ],
    compiler_params=_SC_PARAMS,
)
def _sc_final(agg_hbm, dn_hbm, y_hbm, norm_hbm, b1_hbm, part_out,
              a0v, a1v, yv, nv, e0v, e1v, b1v, accv):
    cid = lax.axis_index("c")
    sid = lax.axis_index("s")
    wid = cid * NS + sid

    pltpu.sync_copy(b1_hbm, b1v)
    accv[...] = jnp.zeros((16,), jnp.float32)

    def chunk(j, carry):
        k = wid + j * NW

        @pl.when(k < NCHK)
        def _():
            base = k * CHK
            pltpu.sync_copy(agg_hbm.at[pl.ds(base, CHK), :], a0v)
            pltpu.sync_copy(agg_hbm.at[pl.ds(N + base, CHK), :], a1v)
            pltpu.sync_copy(y_hbm.at[pl.ds(base, CHK), :], yv)
            pltpu.sync_copy(norm_hbm.at[pl.ds(base, CHK)], nv)
            pltpu.sync_copy(dn_hbm.at[pl.ds(base, CHK)], e0v)
            pltpu.sync_copy(dn_hbm.at[pl.ds(N + base, CHK)], e1v)
            b1r = b1v[...]

            def grp(g, acc):
                sl = pl.ds(g * 16, 16)
                nr = nv[sl]
                w = nr * (e0v[sl] + e1v[sl] - nr)
                for r in range(16):
                    row = a0v[g * 16 + r, :] + a1v[g * 16 + r, :] - yv[g * 16 + r, :]
                    h = jnp.maximum(b1r + nr[r] * row, 0.0)
                    acc = acc + h * w[r]
                return acc

            accv[...] = lax.fori_loop(0, CHK // 16, grp, accv[...])

        return carry

    lax.fori_loop(0, (NCHK + NW - 1) // NW, chunk, 0)
    pltpu.sync_copy(accv, part_out.at[pl.ds(wid * HID_C, HID_C)])


def _tc_head_body(pr, w2r, b2r, war, bar, out):
    s16 = jnp.sum(pr[...], axis=0, keepdims=True)
    sumh2 = jnp.dot(s16, w2r[...], preferred_element_type=jnp.float32)
    feat = (sumh2 + float(N) * b2r[...][None, :]) * (1.0 / float(N))
    out[...] = jnp.dot(feat, war[...], preferred_element_type=jnp.float32) + bar[...][None, :]


def _tc_head(partials, W2, b2, Wa, ba):
    return pl.pallas_call(
        _tc_head_body,
        in_specs=[
            pl.BlockSpec((NW, HID_C), lambda: (0, 0)),
            pl.BlockSpec((HID_C, OUT_C), lambda: (0, 0)),
            pl.BlockSpec((OUT_C,), lambda: (0,)),
            pl.BlockSpec((OUT_C, NUM_OUT), lambda: (0, 0)),
            pl.BlockSpec((NUM_OUT,), lambda: (0,)),
        ],
        out_specs=pl.BlockSpec((1, NUM_OUT), lambda: (0, 0)),
        out_shape=jax.ShapeDtypeStruct((1, NUM_OUT), jnp.float32),
    )(partials, W2, b2, Wa, ba)


@jax.jit
def kernel(x, edge_index, W1, b1, W2, b2, Wa, ba):
    src = edge_index[0]
    dst = edge_index[1]
    zeros_n = jnp.zeros((N,), jnp.float32)
    ones_b = jnp.ones((BA,), jnp.float32)

    zf = _tc_z(x, W1)
    deg_part = _sc_degree(dst, zeros_n, ones_b)
    y, norm = _sc_prep(deg_part, zf)
    agg_part = _sc_agg(src, dst, y)
    dn_part = _sc_dnorm(src, dst, norm)
    partials = _sc_final(agg_part, dn_part, y, norm, b1)
    return _tc_head(partials.reshape(NW, HID_C), W2, b2, Wa, ba)


# speedup vs baseline: 154.0674x; 1.2459x over previous
import functools

import jax
import jax.numpy as jnp
from jax import lax
from jax.experimental import pallas as pl
from jax.experimental.pallas import tpu as pltpu
from jax.experimental.pallas import tpu_sc as plsc

N = 100000
E = 3200000
IN_C = 5
HID_C = 16
OUT_C = 64
NUM_OUT = 10

NC = 2
NS = 16
NW = NC * NS
EW = E // NW
BA = 10000
BC = 1000

_SC_MESH = plsc.VectorSubcoreMesh(core_axis_name="c", subcore_axis_name="s")
_SC_PARAMS = pltpu.CompilerParams(use_tc_tiling_on_sc=False)


@functools.partial(
    pl.kernel,
    out_type=jax.ShapeDtypeStruct((NC * N,), jnp.float32),
    mesh=_SC_MESH,
    scratch_types=[
        pltpu.VMEM((BA,), jnp.int32),
        pltpu.VMEM((BA,), jnp.float32),
        pltpu.VMEM_SHARED((N,), jnp.float32),
    ],
    compiler_params=_SC_PARAMS,
)
def _sc_degree(dst_hbm, zeros_hbm, ones_hbm, deg_out, idx_v, ones_v, deg_sp):
    cid = lax.axis_index("c")
    sid = lax.axis_index("s")
    wid = cid * NS + sid

    @pl.when(sid == 0)
    def _():
        pltpu.sync_copy(zeros_hbm, deg_sp)

    pltpu.sync_copy(ones_hbm, ones_v)
    plsc.subcore_barrier()

    def body(i, carry):
        base = wid * EW + i * BA
        pltpu.sync_copy(dst_hbm.at[pl.ds(base, BA)], idx_v)
        pltpu.sync_copy(ones_v, deg_sp.at[idx_v], add=True)
        return carry

    lax.fori_loop(0, EW // BA, body, 0)
    plsc.subcore_barrier()

    @pl.when(sid < N // BA)
    def _():
        pltpu.sync_copy(deg_sp.at[pl.ds(sid * BA, BA)], ones_v)
        pltpu.sync_copy(ones_v, deg_out.at[pl.ds(cid * N + sid * BA, BA)])


SB = 2400
BCH = 800
NSB_FULL = EW // SB
TAIL = EW - NSB_FULL * SB
NCH = SB // BCH
NCHT = TAIL // BCH


@functools.partial(
    pl.kernel,
    out_type=jax.ShapeDtypeStruct((NC * N, HID_C), jnp.float32),
    mesh=_SC_MESH,
    scratch_types=[
        pltpu.VMEM((SB,), jnp.int32),
        pltpu.VMEM((SB,), jnp.int32),
        pltpu.VMEM((BCH, HID_C), jnp.float32),
        pltpu.VMEM((BCH, HID_C), jnp.float32),
        pltpu.SemaphoreType.DMA,
        pltpu.SemaphoreType.DMA,
        pltpu.SemaphoreType.DMA,
        pltpu.SemaphoreType.DMA,
        pltpu.VMEM_SHARED((N, HID_C), jnp.float32),
    ],
    compiler_params=_SC_PARAMS,
)
def _sc_agg(src_hbm, dst_hbm, y_hbm, agg_out,
            sidx, didx, rows0, rows1, sg0, sg1, ss0, ss1, agg_sp):
    cid = lax.axis_index("c")
    sid = lax.axis_index("s")
    wid = cid * NS + sid
    rows = (rows0, rows1)
    sg = (sg0, sg1)
    ss = (ss0, ss1)

    @pl.when(sid == 0)
    def _():
        pltpu.sync_copy(y_hbm, agg_sp)

    plsc.subcore_barrier()

    def run_sb(base, nchunks):
        l1 = pltpu.async_copy(src_hbm.at[pl.ds(base, nchunks * BCH)],
                              sidx.at[pl.ds(0, nchunks * BCH)], ss[0])
        l2 = pltpu.async_copy(dst_hbm.at[pl.ds(base, nchunks * BCH)],
                              didx.at[pl.ds(0, nchunks * BCH)], ss[1])
        l1.wait()
        l2.wait()
        g = [None, None]
        sc = [None, None]
        g[0] = pltpu.async_copy(y_hbm.at[sidx.at[pl.ds(0, BCH)]], rows[0], sg[0])
        for j in range(nchunks):
            b = j & 1
            nb = b ^ 1
            g[b].wait()
            if j + 1 < nchunks:
                if sc[nb] is not None:
                    sc[nb].wait()
                g[nb] = pltpu.async_copy(
                    y_hbm.at[sidx.at[pl.ds((j + 1) * BCH, BCH)]], rows[nb], sg[nb])
            sc[b] = pltpu.async_copy(
                rows[b], agg_sp.at[didx.at[pl.ds(j * BCH, BCH)]], ss[b], add=True)
        for d in sc:
            if d is not None:
                d.wait()

    def sb_body(s, carry):
        run_sb(wid * EW + s * SB, NCH)
        return carry

    lax.fori_loop(0, NSB_FULL, sb_body, 0)
    run_sb(wid * EW + NSB_FULL * SB, NCHT)
    plsc.subcore_barrier()

    nch = N // BCH

    def wb(j, carry):
        k = sid + j * NS

        @pl.when(k < nch)
        def _():
            pltpu.sync_copy(agg_sp.at[pl.ds(k * BCH, BCH), :], rows0)
            pltpu.sync_copy(rows0, agg_out.at[pl.ds(cid * N + k * BCH, BCH), :])

        return carry

    lax.fori_loop(0, (nch + NS - 1) // NS, wb, 0)


SB2 = 25000
BC2 = 1000
NSB2 = EW // SB2
NCH2 = SB2 // BC2


@functools.partial(
    pl.kernel,
    out_type=jax.ShapeDtypeStruct((NC * N,), jnp.float32),
    mesh=_SC_MESH,
    scratch_types=[
        pltpu.VMEM((SB2,), jnp.int32),
        pltpu.VMEM((SB2,), jnp.int32),
        pltpu.VMEM((BC2,), jnp.float32),
        pltpu.VMEM((BC2,), jnp.float32),
        pltpu.SemaphoreType.DMA,
        pltpu.SemaphoreType.DMA,
        pltpu.SemaphoreType.DMA,
        pltpu.SemaphoreType.DMA,
        pltpu.VMEM_SHARED((N,), jnp.float32),
        pltpu.VMEM_SHARED((N,), jnp.float32),
    ],
    compiler_params=_SC_PARAMS,
)
def _sc_dnorm(src_hbm, dst_hbm, norm_hbm, dn_out,
              sidx, didx, nv0, nv1, sg0, sg1, ss0, ss1, dn_sp, norm_sp):
    cid = lax.axis_index("c")
    sid = lax.axis_index("s")
    wid = cid * NS + sid
    nv = (nv0, nv1)
    sg = (sg0, sg1)
    ss = (ss0, ss1)

    @pl.when(sid == 0)
    def _():
        pltpu.sync_copy(norm_hbm, dn_sp)
        pltpu.sync_copy(norm_hbm, norm_sp)

    plsc.subcore_barrier()

    def sb_body(s, carry):
        base = wid * EW + s * SB2
        pltpu.sync_copy(src_hbm.at[pl.ds(base, SB2)], sidx)
        pltpu.sync_copy(dst_hbm.at[pl.ds(base, SB2)], didx)
        g = [None, None]
        sc = [None, None]
        g[0] = pltpu.async_copy(norm_sp.at[didx.at[pl.ds(0, BC2)]], nv[0], sg[0])
        for j in range(NCH2):
            b = j & 1
            nb = b ^ 1
            g[b].wait()
            if j + 1 < NCH2:
                if sc[nb] is not None:
                    sc[nb].wait()
                g[nb] = pltpu.async_copy(
                    norm_sp.at[didx.at[pl.ds((j + 1) * BC2, BC2)]], nv[nb], sg[nb])
            sc[b] = pltpu.async_copy(
                nv[b], dn_sp.at[sidx.at[pl.ds(j * BC2, BC2)]], ss[b], add=True)
        sc[0].wait()
        sc[1].wait()
        return carry

    lax.fori_loop(0, NSB2, sb_body, 0)
    plsc.subcore_barrier()

    nch = N // BC2

    def wb(j, carry):
        k = sid + j * NS

        @pl.when(k < nch)
        def _():
            pltpu.sync_copy(dn_sp.at[pl.ds(k * BC2, BC2)], nv0)
            pltpu.sync_copy(nv0, dn_out.at[pl.ds(cid * N + k * BC2, BC2)])

        return carry

    lax.fori_loop(0, (nch + NS - 1) // NS, wb, 0)


NBZ = 10000
GZ = N // NBZ


def _tc_z_body(xb, w1, zout):
    zout[...] = jnp.dot(xb[...], w1[...], preferred_element_type=jnp.float32)


def _tc_z(x, W1):
    return pl.pallas_call(
        _tc_z_body,
        grid=(GZ,),
        in_specs=[
            pl.BlockSpec((NBZ, IN_C), lambda i: (i, 0)),
            pl.BlockSpec((IN_C, HID_C), lambda i: (0, 0)),
        ],
        out_specs=pl.BlockSpec((NBZ, HID_C), lambda i: (i, 0)),
        out_shape=jax.ShapeDtypeStruct((N, HID_C), jnp.float32),
    )(x, W1)


CHK = 800
NCHK = N // CHK


def _newton_rsqrt(d):
    i = lax.bitcast_convert_type(d, jnp.int32)
    magic = jnp.full((16,), 0x5F3759DF, jnp.int32)
    r = lax.bitcast_convert_type(magic - (i >> 1), jnp.float32)
    for _ in range(3):
        r = r * (1.5 - 0.5 * d * r * r)
    return r


@functools.partial(
    pl.kernel,
    out_type=[
        jax.ShapeDtypeStruct((N, HID_C), jnp.float32),
        jax.ShapeDtypeStruct((N,), jnp.float32),
    ],
    mesh=_SC_MESH,
    scratch_types=[
        pltpu.VMEM((CHK,), jnp.float32),
        pltpu.VMEM((CHK,), jnp.float32),
        pltpu.VMEM((CHK, HID_C), jnp.float32),
        pltpu.VMEM((CHK,), jnp.float32),
    ],
    compiler_params=_SC_PARAMS,
)
def _sc_prep(degp_hbm, z_hbm, y_out, norm_out, d0v, d1v, zc, nc):
    cid = lax.axis_index("c")
    sid = lax.axis_index("s")
    wid = cid * NS + sid

    def chunk(j, carry):
        k = wid + j * NW

        @pl.when(k < NCHK)
        def _():
            base = k * CHK
            pltpu.sync_copy(degp_hbm.at[pl.ds(base, CHK)], d0v)
            pltpu.sync_copy(degp_hbm.at[pl.ds(N + base, CHK)], d1v)
            pltpu.sync_copy(z_hbm.at[pl.ds(base, CHK), :], zc)

            def grp(g, c2):
                sl = pl.ds(g * 16, 16)
                d = d0v[sl] + d1v[sl] + 1.0
                nr = _newton_rsqrt(d)
                nc[sl] = nr
                for r in range(16):
                    row = zc[g * 16 + r, :]
                    zc[g * 16 + r, :] = row * nr[r]
                return c2

            lax.fori_loop(0, CHK // 16, grp, 0)
            pltpu.sync_copy(nc, norm_out.at[pl.ds(base, CHK)])
            pltpu.sync_copy(zc, y_out.at[pl.ds(base, CHK), :])

        return carry

    lax.fori_loop(0, (NCHK + NW - 1) // NW, chunk, 0)


@functools.partial(
    pl.kernel,
    out_type=jax.ShapeDtypeStruct((NW * HID_C,), jnp.float32),
    mesh=_SC_MESH,
    scratch_types=[
        pltpu.VMEM((CHK, HID_C), jnp.float32),
        pltpu.VMEM((CHK, HID_C), jnp.float32),
        pltpu.VMEM((CHK, HID_C), jnp.float32),
        pltpu.VMEM((CHK,), jnp.float32),
        pltpu.VMEM((CHK,), jnp.float32),
        pltpu.VMEM((CHK,), jnp.float32),
        pltpu.VMEM((16,), jnp.float32),
        pltpu.VMEM((16,), jnp.float32),
    ],
    compiler_params=_SC_PARAMS,
)
def _sc_final(agg_hbm, dn_hbm, y_hbm, norm_hbm, b1_hbm, part_out,
              a0v, a1v, yv, nv, e0v, e1v, b1v, accv):
    cid = lax.axis_index("c")
    sid = lax.axis_index("s")
    wid = cid * NS + sid

    pltpu.sync_copy(b1_hbm, b1v)
    accv[...] = jnp.zeros((16,), jnp.float32)

    def chunk(j, carry):
        k = wid + j * NW

        @pl.when(k < NCHK)
        def _():
            base = k * CHK
            pltpu.sync_copy(agg_hbm.at[pl.ds(base, CHK), :], a0v)
            pltpu.sync_copy(agg_hbm.at[pl.ds(N + base, CHK), :], a1v)
            pltpu.sync_copy(y_hbm.at[pl.ds(base, CHK), :], yv)
            pltpu.sync_copy(norm_hbm.at[pl.ds(base, CHK)], nv)
            pltpu.sync_copy(dn_hbm.at[pl.ds(base, CHK)], e0v)
            pltpu.sync_copy(dn_hbm.at[pl.ds(N + base, CHK)], e1v)
            b1r = b1v[...]

            def grp(g, acc):
                sl = pl.ds(g * 16, 16)
                nr = nv[sl]
                w = nr * (e0v[sl] + e1v[sl] - nr)
                for r in range(16):
                    row = a0v[g * 16 + r, :] + a1v[g * 16 + r, :] - yv[g * 16 + r, :]
                    h = jnp.maximum(b1r + nr[r] * row, 0.0)
                    acc = acc + h * w[r]
                return acc

            accv[...] = lax.fori_loop(0, CHK // 16, grp, accv[...])

        return carry

    lax.fori_loop(0, (NCHK + NW - 1) // NW, chunk, 0)
    pltpu.sync_copy(accv, part_out.at[pl.ds(wid * HID_C, HID_C)])


def _tc_head_body(pr, w2r, b2r, war, bar, out):
    s16 = jnp.sum(pr[...], axis=0, keepdims=True)
    sumh2 = jnp.dot(s16, w2r[...], preferred_element_type=jnp.float32)
    feat = (sumh2 + float(N) * b2r[...][None, :]) * (1.0 / float(N))
    out[...] = jnp.dot(feat, war[...], preferred_element_type=jnp.float32) + bar[...][None, :]


def _tc_head(partials, W2, b2, Wa, ba):
    return pl.pallas_call(
        _tc_head_body,
        in_specs=[
            pl.BlockSpec((NW, HID_C), lambda: (0, 0)),
            pl.BlockSpec((HID_C, OUT_C), lambda: (0, 0)),
            pl.BlockSpec((OUT_C,), lambda: (0,)),
            pl.BlockSpec((OUT_C, NUM_OUT), lambda: (0, 0)),
            pl.BlockSpec((NUM_OUT,), lambda: (0,)),
        ],
        out_specs=pl.BlockSpec((1, NUM_OUT), lambda: (0, 0)),
        out_shape=jax.ShapeDtypeStruct((1, NUM_OUT), jnp.float32),
    )(partials, W2, b2, Wa, ba)


@jax.jit
def kernel(x, edge_index, W1, b1, W2, b2, Wa, ba):
    src = edge_index[0]
    dst = edge_index[1]
    zeros_n = jnp.zeros((N,), jnp.float32)
    ones_b = jnp.ones((BA,), jnp.float32)

    zf = _tc_z(x, W1)
    deg_part = _sc_degree(dst, zeros_n, ones_b)
    y, norm = _sc_prep(deg_part, zf)
    agg_part = _sc_agg(src, dst, y)
    dn_part = _sc_dnorm(src, dst, norm)
    partials = _sc_final(agg_part, dn_part, y, norm, b1)
    return _tc_head(partials.reshape(NW, HID_C), W2, b2, Wa, ba)
